# baseline (speedup 1.0000x reference)
import dataclasses

import jax
import jax.numpy as jnp
from jax import lax
from jax.experimental import pallas as pl
from jax.experimental.pallas import tpu as pltpu
from jax.experimental.pallas import tpu_sc as plsc

_B, _F = 16384, 26
_V, _D = 1000000, 16
_NC, _NS = 2, 16
_NW = _NC * _NS
_RPW = _B // _NW
_C = 64
_NCH = _RPW // _C
_K = _C * _F
_KG = 128
_NG = _K // _KG

_RB = 200
_NB = _V // _RB
_XR = _B // _NW
_XB = 128


def _wid():
    return lax.axis_index("s") * _NC + lax.axis_index("c")


def _repack_body(v_hbm, xv_hbm, xi_hbm, vf_hbm, xvf_hbm, xif_hbm,
                 vbuf, vlin, xvbuf, xvlin, xibuf, xilin, sem):
    wid = _wid()

    @pl.loop(0, (_NB + _NW - 1) // _NW)
    def _blk(t):
        bid = t * _NW + wid

        @pl.when(bid < _NB)
        def _():
            r0 = bid * _RB
            pltpu.sync_copy(v_hbm.at[pl.ds(r0, _RB)], vbuf)

            @pl.loop(0, _RB)
            def _row(r):
                vlin[pl.ds(r * _D, _D)] = vbuf[r, :]

            pltpu.sync_copy(vlin, vf_hbm.at[pl.ds(r0 * _D, _RB * _D)])

    @pl.loop(0, _XR // _XB)
    def _xblk(t):
        x0 = wid * _XR + t * _XB
        pltpu.sync_copy(xv_hbm.at[pl.ds(x0, _XB)], xvbuf)
        pltpu.sync_copy(xi_hbm.at[pl.ds(x0, _XB)], xibuf)

        @pl.loop(0, _XB)
        def _xrow(r):
            xvlin[pl.ds(r * _F, 16)] = xvbuf[r, pl.ds(0, 16)]
            xvlin[pl.ds(r * _F + 10, 16)] = xvbuf[r, pl.ds(10, 16)]
            xilin[pl.ds(r * _F, 16)] = xibuf[r, pl.ds(0, 16)]
            xilin[pl.ds(r * _F + 10, 16)] = xibuf[r, pl.ds(10, 16)]

        pltpu.sync_copy(xvlin, xvf_hbm.at[pl.ds(x0 * _F, _XB * _F)])
        pltpu.sync_copy(xilin, xif_hbm.at[pl.ds(x0 * _F, _XB * _F)])


def _fm_body(xv_hbm, w0_hbm, w_hbm, v_hbm, xi_hbm, out_hbm,
             idx_v, val_v, rows_v, wg_v, out_v, w0_v, sem):
    wid = _wid()
    pltpu.sync_copy(w0_hbm, w0_v)
    w0s = w0_v[...][0]
    lane = lax.iota(jnp.int32, 16)
    ones = jnp.full((16,), 1.0, jnp.float32)
    zeros16 = jnp.zeros((16,), jnp.float32)
    head_mask = jnp.where(lane >= 6, ones, zeros16)

    @pl.loop(0, _NCH)
    def _chunk(c):
        row0 = wid * _RPW + c * _C
        i0 = row0 * _F
        pltpu.sync_copy(xi_hbm.at[pl.ds(i0, _K)], idx_v)
        pltpu.sync_copy(xv_hbm.at[pl.ds(i0, _K)], val_v)
        copies = []
        for j in range(_NG):
            copies.append(pltpu.async_copy(
                v_hbm.at[idx_v.at[pl.ds(j * _KG, _KG)]],
                rows_v.at[pl.ds(j * _KG, _KG)], sem))
        for j in range(_NG):
            copies.append(pltpu.async_copy(
                w_hbm.at[idx_v.at[pl.ds(j * _KG, _KG)]],
                wg_v.at[pl.ds(j * _KG, _KG)], sem))
        for cp2 in copies:
            cp2.wait()

        @pl.loop(0, _C // 16)
        def _group(g):
            res = zeros16
            for l in range(16):
                o = (g * 16 + l) * _F
                va = val_v[pl.ds(o, 16)]
                vb = val_v[pl.ds(o + 10, 16)]
                wa = wg_v[pl.ds(o, 16)]
                wb = wg_v[pl.ds(o + 10, 16)]
                lin = w0s + jnp.sum(va * wa) + jnp.sum(vb * wb * head_mask)
                xv = zeros16
                xsq = zeros16
                for f in range(_F):
                    s = va[f] if f < 16 else vb[f - 10]
                    r = rows_v[o + f, :]
                    p = s * r
                    xv = xv + p
                    xsq = xsq + p * r
                tot = lin + 0.5 * jnp.sum(xv * xv - xsq)
                res = jnp.where(lane == l, tot, res)
            out_v[pl.ds(g * 16, 16)] = res

        pltpu.sync_copy(out_v, out_hbm.at[pl.ds(row0, _C)])


def _mk_params(use_tc_tiling):
    cp = pltpu.CompilerParams()
    fields = pltpu.CompilerParams.__dataclass_fields__
    if "needs_layout_passes" in fields:
        cp = dataclasses.replace(cp, needs_layout_passes=False)
    if "use_tc_tiling_on_sc" in fields:
        cp = dataclasses.replace(cp, use_tc_tiling_on_sc=use_tc_tiling)
    return cp


def kernel(x_val, w0, w, v, x_idx):
    mesh = plsc.VectorSubcoreMesh(core_axis_name="c", subcore_axis_name="s")
    repack = pl.kernel(
        _repack_body,
        out_type=(
            jax.ShapeDtypeStruct((_V * _D,), jnp.float32),
            jax.ShapeDtypeStruct((_B * _F,), jnp.float32),
            jax.ShapeDtypeStruct((_B * _F,), jnp.int32),
        ),
        mesh=mesh,
        compiler_params=_mk_params(True),
        scratch_types=[
            pltpu.VMEM((_RB, _D), jnp.float32),
            pltpu.VMEM((_RB * _D,), jnp.float32),
            pltpu.VMEM((_XB, _F), jnp.float32),
            pltpu.VMEM((_XB * _F,), jnp.float32),
            pltpu.VMEM((_XB, _F), jnp.int32),
            pltpu.VMEM((_XB * _F,), jnp.int32),
            pltpu.SemaphoreType.DMA,
        ],
    )
    fm = pl.kernel(
        _fm_body,
        out_type=jax.ShapeDtypeStruct((_B,), jnp.float32),
        mesh=mesh,
        compiler_params=_mk_params(False),
        scratch_types=[
            pltpu.VMEM((_K,), jnp.int32),
            pltpu.VMEM((_K,), jnp.float32),
            pltpu.VMEM((_K, _D), jnp.float32),
            pltpu.VMEM((_K,), jnp.float32),
            pltpu.VMEM((_C,), jnp.float32),
            pltpu.VMEM((16,), jnp.float32),
            pltpu.SemaphoreType.DMA,
        ],
    )
    v_flat, xv_flat, xi_flat = repack(v, x_val, x_idx.astype(jnp.int32))
    return fm(xv_flat, jnp.tile(w0, 16), w, v_flat.reshape(_V, _D), xi_flat)


# speedup vs baseline: 1.1495x; 1.1495x over previous
import dataclasses

import jax
import jax.numpy as jnp
from jax import lax
from jax.experimental import pallas as pl
from jax.experimental.pallas import tpu as pltpu
from jax.experimental.pallas import tpu_sc as plsc

_B, _F = 16384, 26
_V, _D = 1000000, 16
_NC, _NS = 2, 16
_NW = _NC * _NS
_RPW = _B // _NW
_C = 64
_NCH = _RPW // _C
_K = _C * _F
_KG = 128
_NG = _K // _KG

_RB = 200
_NB = _V // _RB
_NBUF = 4
_RND = _NB // _NW
_ITER = _RND // _NBUF


def _wid():
    return lax.axis_index("s") * _NC + lax.axis_index("c")


def _compact_rows(dst, src, n):
    @pl.loop(0, n)
    def _r(r):
        dst[pl.ds(r * _F, 16)] = src[r, pl.ds(0, 16)]
        dst[pl.ds(r * _F + 10, 16)] = src[r, pl.ds(10, 16)]


def _repack_body(v_hbm, vf_hbm, b0, b1, b2, b3, l0, l1, l2, l3,
                 s0, s1, s2, s3, so0, so1, so2, so3):
    wid = _wid()
    bufs = (b0, b1, b2, b3)
    lins = (l0, l1, l2, l3)
    sin = (s0, s1, s2, s3)
    sout = (so0, so1, so2, so3)

    @pl.loop(0, _ITER)
    def _t(t):
        rnd0 = t * _NBUF
        hin = []
        for p in range(_NBUF):
            bid = (rnd0 + p) * _NW + wid
            hin.append(pltpu.async_copy(
                v_hbm.at[pl.ds(bid * _RB, _RB)], bufs[p], sin[p]))
        hout = []
        for p in range(_NBUF):
            bid = (rnd0 + p) * _NW + wid
            hin[p].wait()

            @pl.loop(0, _RB)
            def _row(r, _p=p):
                lins[_p][pl.ds(r * _D, _D)] = bufs[_p][r, :]

            hout.append(pltpu.async_copy(
                lins[p], vf_hbm.at[pl.ds(bid * _RB * _D, _RB * _D)],
                sout[p]))
        for h in hout:
            h.wait()

    @pl.when(wid < _NB - _RND * _NW)
    def _tail():
        bid = _RND * _NW + wid
        pltpu.sync_copy(v_hbm.at[pl.ds(bid * _RB, _RB)], b0)

        @pl.loop(0, _RB)
        def _row(r):
            l0[pl.ds(r * _D, _D)] = b0[r, :]

        pltpu.sync_copy(l0, vf_hbm.at[pl.ds(bid * _RB * _D, _RB * _D)])


def _fm_body(xv_hbm, w0_hbm, w_hbm, v_hbm, xi_hbm, out_hbm,
             idx2, val2, idx_v, val_v, rows_v, wg_v, out_v, w0_v, sem):
    wid = _wid()
    pltpu.sync_copy(w0_hbm, w0_v)
    w0s = w0_v[...][0]
    lane = lax.iota(jnp.int32, 16)
    ones = jnp.full((16,), 1.0, jnp.float32)
    zeros16 = jnp.zeros((16,), jnp.float32)
    head_mask = jnp.where(lane >= 6, ones, zeros16)

    @pl.loop(0, _NCH)
    def _chunk(c):
        row0 = wid * _RPW + c * _C
        pltpu.sync_copy(xi_hbm.at[pl.ds(row0, _C)], idx2)
        pltpu.sync_copy(xv_hbm.at[pl.ds(row0, _C)], val2)
        _compact_rows(idx_v, idx2, _C)
        _compact_rows(val_v, val2, _C)
        copies = []
        for j in range(_NG):
            copies.append(pltpu.async_copy(
                v_hbm.at[idx_v.at[pl.ds(j * _KG, _KG)]],
                rows_v.at[pl.ds(j * _KG, _KG)], sem))
        for j in range(_NG):
            copies.append(pltpu.async_copy(
                w_hbm.at[idx_v.at[pl.ds(j * _KG, _KG)]],
                wg_v.at[pl.ds(j * _KG, _KG)], sem))
        for cp2 in copies:
            cp2.wait()

        @pl.loop(0, _C // 16)
        def _group(g):
            res = zeros16
            for l in range(16):
                o = (g * 16 + l) * _F
                va = val_v[pl.ds(o, 16)]
                vb = val_v[pl.ds(o + 10, 16)]
                wa = wg_v[pl.ds(o, 16)]
                wb = wg_v[pl.ds(o + 10, 16)]
                lin = w0s + jnp.sum(va * wa) + jnp.sum(vb * wb * head_mask)
                xv = zeros16
                xsq = zeros16
                for f in range(_F):
                    s = va[f] if f < 16 else vb[f - 10]
                    r = rows_v[o + f, :]
                    p = s * r
                    xv = xv + p
                    xsq = xsq + p * r
                tot = lin + 0.5 * jnp.sum(xv * xv - xsq)
                res = jnp.where(lane == l, tot, res)
            out_v[pl.ds(g * 16, 16)] = res

        pltpu.sync_copy(out_v, out_hbm.at[pl.ds(row0, _C)])


def _mk_params(use_tc_tiling):
    cp = pltpu.CompilerParams()
    fields = pltpu.CompilerParams.__dataclass_fields__
    if "needs_layout_passes" in fields:
        cp = dataclasses.replace(cp, needs_layout_passes=False)
    if "use_tc_tiling_on_sc" in fields:
        cp = dataclasses.replace(cp, use_tc_tiling_on_sc=use_tc_tiling)
    return cp


def kernel(x_val, w0, w, v, x_idx):
    mesh = plsc.VectorSubcoreMesh(core_axis_name="c", subcore_axis_name="s")
    repack = pl.kernel(
        _repack_body,
        out_type=jax.ShapeDtypeStruct((_V * _D,), jnp.float32),
        mesh=mesh,
        compiler_params=_mk_params(True),
        scratch_types=(
            [pltpu.VMEM((_RB, _D), jnp.float32) for _ in range(_NBUF)]
            + [pltpu.VMEM((_RB * _D,), jnp.float32) for _ in range(_NBUF)]
            + [pltpu.SemaphoreType.DMA for _ in range(2 * _NBUF)]
        ),
    )
    fm = pl.kernel(
        _fm_body,
        out_type=jax.ShapeDtypeStruct((_B,), jnp.float32),
        mesh=mesh,
        compiler_params=_mk_params(False),
        scratch_types=[
            pltpu.VMEM((_C, _F), jnp.int32),
            pltpu.VMEM((_C, _F), jnp.float32),
            pltpu.VMEM((_K,), jnp.int32),
            pltpu.VMEM((_K,), jnp.float32),
            pltpu.VMEM((_K, _D), jnp.float32),
            pltpu.VMEM((_K,), jnp.float32),
            pltpu.VMEM((_C,), jnp.float32),
            pltpu.VMEM((16,), jnp.float32),
            pltpu.SemaphoreType.DMA,
        ],
    )
    v_flat = repack(v)
    return fm(x_val, jnp.tile(w0, 16), w, v_flat.reshape(_V, _D),
              x_idx.astype(jnp.int32))


# speedup vs baseline: 1.5760x; 1.3710x over previous
import dataclasses

import jax
import jax.numpy as jnp
from jax import lax
from jax.experimental import pallas as pl
from jax.experimental.pallas import tpu as pltpu
from jax.experimental.pallas import tpu_sc as plsc

_B, _F = 16384, 26
_V, _D = 1000000, 16
_NC, _NS = 2, 16
_NW = _NC * _NS
_RPW = _B // _NW
_C = 64
_NCH = _RPW // _C
_K = _C * _F
_KG = 128
_NG = _K // _KG


def _fm_body(x_val_hbm, w0_hbm, w_hbm, v_hbm, x_idx_hbm, out_hbm,
             idx_v, val_v, rows_v, wg_v, out_v, w0_v, sem):
    wid = lax.axis_index("s") * _NC + lax.axis_index("c")
    pltpu.sync_copy(w0_hbm, w0_v)
    w0s = w0_v[...][0]
    lane = lax.iota(jnp.int32, 16)
    ones = jnp.full((16,), 1.0, jnp.float32)
    zeros16 = jnp.zeros((16,), jnp.float32)
    head_mask = jnp.where(lane >= 6, ones, zeros16)

    @pl.loop(0, _NCH)
    def _chunk(c):
        row0 = wid * _RPW + c * _C
        i0 = row0 * _F
        pltpu.sync_copy(x_idx_hbm.at[pl.ds(i0, _K)], idx_v)
        pltpu.sync_copy(x_val_hbm.at[pl.ds(i0, _K)], val_v)
        copies = []
        for j in range(_NG):
            copies.append(pltpu.async_copy(
                v_hbm.at[idx_v.at[pl.ds(j * _KG, _KG)]],
                rows_v.at[pl.ds(j * _KG, _KG)], sem))
        for j in range(_NG):
            copies.append(pltpu.async_copy(
                w_hbm.at[idx_v.at[pl.ds(j * _KG, _KG)]],
                wg_v.at[pl.ds(j * _KG, _KG)], sem))
        for cp2 in copies:
            cp2.wait()

        @pl.loop(0, _C // 16)
        def _group(g):
            res = zeros16
            for l in range(16):
                o = (g * 16 + l) * _F
                va = val_v[pl.ds(o, 16)]
                vb = val_v[pl.ds(o + 10, 16)]
                wa = wg_v[pl.ds(o, 16)]
                wb = wg_v[pl.ds(o + 10, 16)]
                lin = w0s + jnp.sum(va * wa) + jnp.sum(vb * wb * head_mask)
                xv = zeros16
                xsq = zeros16
                for f in range(_F):
                    s = va[f] if f < 16 else vb[f - 10]
                    r = rows_v[o + f, :]
                    p = s * r
                    xv = xv + p
                    xsq = xsq + p * r
                tot = lin + 0.5 * jnp.sum(xv * xv - xsq)
                res = jnp.where(lane == l, tot, res)
            out_v[pl.ds(g * 16, 16)] = res

        pltpu.sync_copy(out_v, out_hbm.at[pl.ds(row0, _C)])


def kernel(x_val, w0, w, v, x_idx):
    idx_flat = x_idx.astype(jnp.int32).reshape(_B * _F)
    val_flat = x_val.reshape(_B * _F)
    mesh = plsc.VectorSubcoreMesh(core_axis_name="c", subcore_axis_name="s")
    cp = pltpu.CompilerParams()
    if "needs_layout_passes" in pltpu.CompilerParams.__dataclass_fields__:
        cp = dataclasses.replace(cp, needs_layout_passes=False)
    if "use_tc_tiling_on_sc" in pltpu.CompilerParams.__dataclass_fields__:
        cp = dataclasses.replace(cp, use_tc_tiling_on_sc=False)
    fm = pl.kernel(
        _fm_body,
        out_type=jax.ShapeDtypeStruct((_B,), jnp.float32),
        mesh=mesh,
        compiler_params=cp,
        scratch_types=[
            pltpu.VMEM((_K,), jnp.int32),
            pltpu.VMEM((_K,), jnp.float32),
            pltpu.VMEM((_K, _D), jnp.float32),
            pltpu.VMEM((_K,), jnp.float32),
            pltpu.VMEM((_C,), jnp.float32),
            pltpu.VMEM((16,), jnp.float32),
            pltpu.SemaphoreType.DMA,
        ],
    )
    return fm(val_flat, jnp.tile(w0, 16), w, v, idx_flat)
